# baseline (speedup 1.0000x reference)
import functools

import jax
import jax.numpy as jnp
from jax import lax
from jax.experimental import pallas as pl
from jax.experimental.pallas import tpu as pltpu
from jax.experimental.pallas import tpu_sc as plsc

NUM_CORES = 2
NUM_SUBCORES = 16
NUM_WORKERS = NUM_CORES * NUM_SUBCORES
LANES = 16

BATCH = 16384
EMBED_DIM = 64
ROW = 2 * EMBED_DIM

N_SPLITS = 2
SPLIT = BATCH // N_SPLITS
B_PER_W = SPLIT // NUM_WORKERS
CHUNK = 128
N_CHUNKS = B_PER_W // CHUNK

TC_GRID = 4
TC_BLOCK = SPLIT // TC_GRID


def _gather_body(heads_r, rels_r, tails_r, ent_r, rel_r,
                 hg_r, tg_r, rg_r,
                 hidx_v, ridx_v, tidx_v, h_v, t_v, r_v,
                 sem_h, sem_t, sem_r, sem_w):
    wid = lax.axis_index("s") * NUM_CORES + lax.axis_index("c")

    pltpu.sync_copy(heads_r.at[wid], hidx_v)
    pltpu.sync_copy(rels_r.at[wid], ridx_v)
    pltpu.sync_copy(tails_r.at[wid], tidx_v)

    def issue_in(j, slot):
        ch = pltpu.async_copy(ent_r.at[hidx_v.at[j]], h_v.at[slot], sem_h)
        ct = pltpu.async_copy(ent_r.at[tidx_v.at[j]], t_v.at[slot], sem_t)
        cr = pltpu.async_copy(rel_r.at[ridx_v.at[j]], r_v.at[slot], sem_r)
        return ch, ct, cr

    pending = issue_in(0, 0)
    writes = [None, None]
    for j in range(N_CHUNKS):
        slot = j & 1
        for c in pending:
            c.wait()
        if j + 1 < N_CHUNKS:
            other = (j + 1) & 1
            if writes[other] is not None:
                for c in writes[other]:
                    c.wait()
            pending = issue_in(j + 1, other)
        wh = pltpu.async_copy(h_v.at[slot], hg_r.at[wid, j], sem_w)
        wt = pltpu.async_copy(t_v.at[slot], tg_r.at[wid, j], sem_w)
        wr = pltpu.async_copy(r_v.at[slot], rg_r.at[wid, j], sem_w)
        writes[slot] = (wh, wt, wr)
    for w in writes:
        if w is not None:
            for c in w:
                c.wait()


def _tc_body(h_ref, t_ref, r_ref, o_ref):
    h = h_ref[...]
    t = t_ref[...]
    r = r_ref[:, :EMBED_DIM]
    hre, him = h[:, :EMBED_DIM], h[:, EMBED_DIM:]
    tre, tim = t[:, :EMBED_DIM], t[:, EMBED_DIM:]
    cr = jnp.cos(r)
    sr = jnp.sin(r)
    dre = hre * cr - him * sr - tre
    dim = hre * sr + him * cr - tim
    sc = jnp.sqrt(dre * dre + dim * dim).sum(axis=-1)
    o_ref[...] = sc.reshape(o_ref.shape)


@jax.jit
def _rotate_score(heads, relations, tails, entity_emb, relation_emb):
    mesh = plsc.VectorSubcoreMesh(
        core_axis_name="c", subcore_axis_name="s",
        num_cores=NUM_CORES, num_subcores=NUM_SUBCORES)
    row_t = jax.ShapeDtypeStruct((NUM_WORKERS, N_CHUNKS, CHUNK, ROW),
                                 jnp.float32)
    sc_gather = pl.kernel(
        _gather_body,
        out_type=(row_t, row_t, row_t),
        mesh=mesh,
        scratch_types=[
            pltpu.VMEM((N_CHUNKS, CHUNK), jnp.int32),
            pltpu.VMEM((N_CHUNKS, CHUNK), jnp.int32),
            pltpu.VMEM((N_CHUNKS, CHUNK), jnp.int32),
            pltpu.VMEM((2, CHUNK, ROW), jnp.float32),
            pltpu.VMEM((2, CHUNK, ROW), jnp.float32),
            pltpu.VMEM((2, CHUNK, ROW), jnp.float32),
            pltpu.SemaphoreType.DMA,
            pltpu.SemaphoreType.DMA,
            pltpu.SemaphoreType.DMA,
            pltpu.SemaphoreType.DMA,
        ],
    )

    tc_score = pl.pallas_call(
        _tc_body,
        grid=(TC_GRID,),
        in_specs=[
            pl.BlockSpec((TC_BLOCK, ROW), lambda i: (i, 0)),
            pl.BlockSpec((TC_BLOCK, ROW), lambda i: (i, 0)),
            pl.BlockSpec((TC_BLOCK, ROW), lambda i: (i, 0)),
        ],
        out_specs=pl.BlockSpec((TC_BLOCK // 128, 128), lambda i: (i, 0)),
        out_shape=jax.ShapeDtypeStruct((SPLIT // 128, 128), jnp.float32),
    )

    outs = []
    for s in range(N_SPLITS):
        hg, tg, rg = sc_gather(heads[s], relations[s], tails[s],
                               entity_emb, relation_emb)
        outs.append((hg.reshape(SPLIT, ROW), tg.reshape(SPLIT, ROW),
                     rg.reshape(SPLIT, ROW)))
    scores = [tc_score(hg, tg, rg).reshape(SPLIT) for hg, tg, rg in outs]
    return jnp.concatenate(scores)


def kernel(heads, relations, tails, entity_emb, relation_emb):
    shp = (N_SPLITS, NUM_WORKERS, N_CHUNKS, CHUNK)
    heads = heads.astype(jnp.int32).reshape(shp)
    relations = relations.astype(jnp.int32).reshape(shp)
    tails = tails.astype(jnp.int32).reshape(shp)
    relation_emb = jnp.pad(relation_emb, ((0, 0), (0, EMBED_DIM)))
    return _rotate_score(heads, relations, tails, entity_emb, relation_emb)


# speedup vs baseline: 1.9373x; 1.9373x over previous
import functools

import jax
import jax.numpy as jnp
from jax import lax
from jax.experimental import pallas as pl
from jax.experimental.pallas import tpu as pltpu
from jax.experimental.pallas import tpu_sc as plsc

NUM_CORES = 2
NUM_SUBCORES = 16
NUM_WORKERS = NUM_CORES * NUM_SUBCORES
LANES = 16

BATCH = 16384
EMBED_DIM = 64
ROW = 2 * EMBED_DIM
NUM_RELS = 1000
B_PER_W = BATCH // NUM_WORKERS
CHUNK = 128
N_CHUNKS = B_PER_W // CHUNK


def _f32(x):
    return jnp.float32(x)


_GATHER_DNUMS = lax.GatherDimensionNumbers(
    offset_dims=(), collapsed_slice_dims=(0,), start_index_map=(0,))


def _shuffle(x, idx):
    return lax.gather(
        x, idx[:, None], dimension_numbers=_GATHER_DNUMS, slice_sizes=(1,),
        mode=lax.GatherScatterMode.PROMISE_IN_BOUNDS)


def _sqrt16(s):
    s = s + _f32(1e-35)
    i = lax.bitcast_convert_type(s, jnp.int32)
    i = jnp.int32(0x5F3759DF) - lax.shift_right_logical(i, jnp.int32(1))
    y = lax.bitcast_convert_type(i, jnp.float32)
    y = y * (_f32(1.5) - _f32(0.5) * s * y * y)
    return s * y


def _cs_table_body(rel_ref, out_ref):
    r = rel_ref[...]
    out_ref[:, :EMBED_DIM] = jnp.cos(r)
    out_ref[:, EMBED_DIM:] = jnp.sin(r)


def _score_body(heads_r, rels_r, tails_r, ent_r, cs_r, out_r,
                hidx_v, ridx_v, tidx_v, h_v, t_v, r_v, out_v,
                sem_h, sem_t, sem_r):
    wid = lax.axis_index("s") * NUM_CORES + lax.axis_index("c")

    pltpu.sync_copy(heads_r.at[wid], hidx_v)
    pltpu.sync_copy(rels_r.at[wid], ridx_v)
    pltpu.sync_copy(tails_r.at[wid], tidx_v)

    def issue(j, slot):
        ch = pltpu.async_copy(ent_r.at[hidx_v.at[j]], h_v.at[slot], sem_h)
        ct = pltpu.async_copy(ent_r.at[tidx_v.at[j]], t_v.at[slot], sem_t)
        cr = pltpu.async_copy(cs_r.at[ridx_v.at[j]], r_v.at[slot], sem_r)
        return ch, ct, cr

    pending = issue(0, 0)
    lane = lax.iota(jnp.int32, LANES)
    lane_masks = [lane == jnp.int32(i) for i in range(LANES)]

    for j in range(N_CHUNKS):
        slot = j & 1
        for c in pending:
            c.wait()
        if j + 1 < N_CHUNKS:
            pending = issue(j + 1, (j + 1) & 1)

        def group_body(g, _):
            res = jnp.zeros((LANES,), jnp.float32)
            for i in range(LANES):
                b = g * LANES + i
                acc = jnp.zeros((LANES,), jnp.float32)
                for k in range(EMBED_DIM // LANES):
                    cosr = r_v[slot, b, pl.ds(k * LANES, LANES)]
                    sinr = r_v[slot, b, pl.ds(EMBED_DIM + k * LANES, LANES)]
                    hre = h_v[slot, b, pl.ds(k * LANES, LANES)]
                    him = h_v[slot, b, pl.ds(EMBED_DIM + k * LANES, LANES)]
                    tre = t_v[slot, b, pl.ds(k * LANES, LANES)]
                    tim = t_v[slot, b, pl.ds(EMBED_DIM + k * LANES, LANES)]
                    dre = hre * cosr - him * sinr - tre
                    dim = hre * sinr + him * cosr - tim
                    acc = acc + _sqrt16(dre * dre + dim * dim)
                for m in (1, 2, 4, 8):
                    acc = acc + _shuffle(acc, lane ^ m)
                res = lax.select(lane_masks[i], acc, res)
            out_v[pl.ds(j * CHUNK + g * LANES, LANES)] = res
            return _

        lax.fori_loop(0, CHUNK // LANES, group_body, None)

    pltpu.sync_copy(out_v, out_r.at[wid])


@jax.jit
def _rotate_score(heads, relations, tails, entity_emb, relation_emb):
    cs_table = pl.pallas_call(
        _cs_table_body,
        out_shape=jax.ShapeDtypeStruct((NUM_RELS, ROW), jnp.float32),
    )(relation_emb)

    mesh = plsc.VectorSubcoreMesh(
        core_axis_name="c", subcore_axis_name="s",
        num_cores=NUM_CORES, num_subcores=NUM_SUBCORES)
    run = pl.kernel(
        _score_body,
        out_type=jax.ShapeDtypeStruct((NUM_WORKERS, B_PER_W), jnp.float32),
        mesh=mesh,
        scratch_types=[
            pltpu.VMEM((N_CHUNKS, CHUNK), jnp.int32),
            pltpu.VMEM((N_CHUNKS, CHUNK), jnp.int32),
            pltpu.VMEM((N_CHUNKS, CHUNK), jnp.int32),
            pltpu.VMEM((2, CHUNK, ROW), jnp.float32),
            pltpu.VMEM((2, CHUNK, ROW), jnp.float32),
            pltpu.VMEM((2, CHUNK, ROW), jnp.float32),
            pltpu.VMEM((B_PER_W,), jnp.float32),
            pltpu.SemaphoreType.DMA,
            pltpu.SemaphoreType.DMA,
            pltpu.SemaphoreType.DMA,
        ],
    )
    out = run(heads, relations, tails, entity_emb, cs_table)
    return out.reshape(BATCH)


def kernel(heads, relations, tails, entity_emb, relation_emb):
    heads = heads.astype(jnp.int32).reshape(NUM_WORKERS, N_CHUNKS, CHUNK)
    relations = relations.astype(jnp.int32).reshape(NUM_WORKERS, N_CHUNKS, CHUNK)
    tails = tails.astype(jnp.int32).reshape(NUM_WORKERS, N_CHUNKS, CHUNK)
    return _rotate_score(heads, relations, tails, entity_emb, relation_emb)


# speedup vs baseline: 1.9840x; 1.0241x over previous
import jax
import jax.numpy as jnp
from jax import lax
from jax.experimental import pallas as pl
from jax.experimental.pallas import tpu as pltpu
from jax.experimental.pallas import tpu_sc as plsc

NUM_CORES = 2
NUM_SUBCORES = 16
NUM_WORKERS = NUM_CORES * NUM_SUBCORES
LANES = 16

BATCH = 16384
EMBED_DIM = 64
ROW = 2 * EMBED_DIM
NUM_RELS = 1000
B_PER_W = BATCH // NUM_WORKERS
CHUNK = 128
N_CHUNKS = B_PER_W // CHUNK


def _f32(x):
    return jnp.float32(x)


_GATHER_DNUMS = lax.GatherDimensionNumbers(
    offset_dims=(), collapsed_slice_dims=(0,), start_index_map=(0,))


def _shuffle(x, idx):
    return lax.gather(
        x, idx[:, None], dimension_numbers=_GATHER_DNUMS, slice_sizes=(1,),
        mode=lax.GatherScatterMode.PROMISE_IN_BOUNDS)


def _sqrt16(s):
    s = s + _f32(1e-35)
    i = lax.bitcast_convert_type(s, jnp.int32)
    i = jnp.int32(0x5F3759DF) - lax.shift_right_logical(i, jnp.int32(1))
    y = lax.bitcast_convert_type(i, jnp.float32)
    y = y * (_f32(1.5) - _f32(0.5) * s * y * y)
    return s * y


def _cs_table_body(rel_ref, out_ref):
    r = rel_ref[...]
    out_ref[:, :EMBED_DIM] = jnp.cos(r)
    out_ref[:, EMBED_DIM:] = jnp.sin(r)


def _score_body(idx_r, ent_r, cs_r, out_r,
                idx_v, h_v, t_v, r_v, out_v,
                sem_h, sem_t, sem_r):
    wid = lax.axis_index("s") * NUM_CORES + lax.axis_index("c")

    pltpu.sync_copy(idx_r.at[wid], idx_v)

    def issue(j, slot):
        ch = pltpu.async_copy(ent_r.at[idx_v.at[0, j]], h_v.at[slot], sem_h)
        ct = pltpu.async_copy(ent_r.at[idx_v.at[2, j]], t_v.at[slot], sem_t)
        cr = pltpu.async_copy(cs_r.at[idx_v.at[1, j]], r_v.at[slot], sem_r)
        return ch, ct, cr

    pending = issue(0, 0)
    lane = lax.iota(jnp.int32, LANES)
    tree_sels = {m: (lane & m) != 0 for m in (1, 2, 4, 8)}
    tree_perms = {m: lane ^ m for m in (1, 2, 4, 8)}

    for j in range(N_CHUNKS):
        slot = j & 1
        for c in pending:
            c.wait()
        if j + 1 < N_CHUNKS:
            pending = issue(j + 1, (j + 1) & 1)

        def group_body(g, _):
            accs = []
            for i in range(LANES):
                b = g * LANES + i
                acc = jnp.zeros((LANES,), jnp.float32)
                for k in range(EMBED_DIM // LANES):
                    cosr = r_v[slot, b, pl.ds(k * LANES, LANES)]
                    sinr = r_v[slot, b, pl.ds(EMBED_DIM + k * LANES, LANES)]
                    hre = h_v[slot, b, pl.ds(k * LANES, LANES)]
                    him = h_v[slot, b, pl.ds(EMBED_DIM + k * LANES, LANES)]
                    tre = t_v[slot, b, pl.ds(k * LANES, LANES)]
                    tim = t_v[slot, b, pl.ds(EMBED_DIM + k * LANES, LANES)]
                    dre = hre * cosr - him * sinr - tre
                    dim = hre * sinr + him * cosr - tim
                    acc = acc + _sqrt16(dre * dre + dim * dim)
                accs.append(acc)
            for m in (1, 2, 4, 8):
                sel, perm = tree_sels[m], tree_perms[m]
                accs = [
                    lax.select(sel, accs[i + 1], accs[i])
                    + _shuffle(lax.select(sel, accs[i], accs[i + 1]), perm)
                    for i in range(0, len(accs), 2)
                ]
            out_v[pl.ds(j * CHUNK + g * LANES, LANES)] = accs[0]
            return _

        lax.fori_loop(0, CHUNK // LANES, group_body, None)

    pltpu.sync_copy(out_v, out_r.at[pl.ds(wid * B_PER_W, B_PER_W)])


@jax.jit
def _rotate_score(idx, entity_emb, relation_emb):
    cs_table = pl.pallas_call(
        _cs_table_body,
        out_shape=jax.ShapeDtypeStruct((NUM_RELS, ROW), jnp.float32),
    )(relation_emb)

    mesh = plsc.VectorSubcoreMesh(
        core_axis_name="c", subcore_axis_name="s",
        num_cores=NUM_CORES, num_subcores=NUM_SUBCORES)
    run = pl.kernel(
        _score_body,
        out_type=jax.ShapeDtypeStruct((BATCH,), jnp.float32),
        mesh=mesh,
        scratch_types=[
            pltpu.VMEM((3, N_CHUNKS, CHUNK), jnp.int32),
            pltpu.VMEM((2, CHUNK, ROW), jnp.float32),
            pltpu.VMEM((2, CHUNK, ROW), jnp.float32),
            pltpu.VMEM((2, CHUNK, ROW), jnp.float32),
            pltpu.VMEM((B_PER_W,), jnp.float32),
            pltpu.SemaphoreType.DMA,
            pltpu.SemaphoreType.DMA,
            pltpu.SemaphoreType.DMA,
        ],
    )
    return run(idx, entity_emb, cs_table)


def kernel(heads, relations, tails, entity_emb, relation_emb):
    idx = jnp.stack([heads.astype(jnp.int32), relations.astype(jnp.int32),
                     tails.astype(jnp.int32)], axis=0)
    idx = idx.reshape(3, NUM_WORKERS, N_CHUNKS, CHUNK).transpose(1, 0, 2, 3)
    return _rotate_score(idx, entity_emb, relation_emb)


# speedup vs baseline: 2.0649x; 1.0408x over previous
import jax
import jax.numpy as jnp
from jax import lax
from jax.experimental import pallas as pl
from jax.experimental.pallas import tpu as pltpu
from jax.experimental.pallas import tpu_sc as plsc

NUM_CORES = 2
NUM_SUBCORES = 16
NUM_WORKERS = NUM_CORES * NUM_SUBCORES
LANES = 16

BATCH = 16384
EMBED_DIM = 64
ROW = 2 * EMBED_DIM
NUM_RELS = 1000
B_PER_W = BATCH // NUM_WORKERS
CHUNK = 128
N_CHUNKS = B_PER_W // CHUNK


def _f32(x):
    return jnp.float32(x)


_GATHER_DNUMS = lax.GatherDimensionNumbers(
    offset_dims=(), collapsed_slice_dims=(0,), start_index_map=(0,))


def _shuffle(x, idx):
    return lax.gather(
        x, idx[:, None], dimension_numbers=_GATHER_DNUMS, slice_sizes=(1,),
        mode=lax.GatherScatterMode.PROMISE_IN_BOUNDS)


def _sqrt16(s):
    s = s + _f32(1e-35)
    i = lax.bitcast_convert_type(s, jnp.int32)
    i = jnp.int32(0x5F3759DF) - lax.shift_right_logical(i, jnp.int32(1))
    y = lax.bitcast_convert_type(i, jnp.float32)
    y = y * (_f32(1.5) - _f32(0.5) * s * y * y)
    return s * y


def _cs_table_body(rel_ref, out_ref):
    r = rel_ref[...]
    out_ref[:, :EMBED_DIM] = jnp.cos(r)
    out_ref[:, EMBED_DIM:] = jnp.sin(r)


def _score_body(heads_r, rels_r, tails_r, ent_r, cs_r, out_r,
                hidx_v, ridx_v, tidx_v, h_v, t_v, r_v, out_v,
                sem_h, sem_t, sem_r):
    wid = lax.axis_index("s") * NUM_CORES + lax.axis_index("c")

    c1 = pltpu.async_copy(heads_r.at[wid], hidx_v, sem_h)
    c2 = pltpu.async_copy(rels_r.at[wid], ridx_v, sem_r)
    c3 = pltpu.async_copy(tails_r.at[wid], tidx_v, sem_t)
    c1.wait()
    c2.wait()
    c3.wait()

    def issue(j, slot):
        ch = pltpu.async_copy(ent_r.at[hidx_v.at[j]], h_v.at[slot], sem_h)
        ct = pltpu.async_copy(ent_r.at[tidx_v.at[j]], t_v.at[slot], sem_t)
        cr = pltpu.async_copy(cs_r.at[ridx_v.at[j]], r_v.at[slot], sem_r)
        return ch, ct, cr

    pending = issue(0, 0)
    lane = lax.iota(jnp.int32, LANES)
    tree_sels = {m: (lane & m) != 0 for m in (1, 2, 4, 8)}
    tree_perms = {m: lane ^ m for m in (1, 2, 4, 8)}

    for j in range(N_CHUNKS):
        slot = j & 1
        for c in pending:
            c.wait()
        if j + 1 < N_CHUNKS:
            pending = issue(j + 1, (j + 1) & 1)

        def group_body(g, _):
            accs = []
            for i in range(LANES):
                b = g * LANES + i
                acc = jnp.zeros((LANES,), jnp.float32)
                for k in range(EMBED_DIM // LANES):
                    cosr = r_v[slot, b, pl.ds(k * LANES, LANES)]
                    sinr = r_v[slot, b, pl.ds(EMBED_DIM + k * LANES, LANES)]
                    hre = h_v[slot, b, pl.ds(k * LANES, LANES)]
                    him = h_v[slot, b, pl.ds(EMBED_DIM + k * LANES, LANES)]
                    tre = t_v[slot, b, pl.ds(k * LANES, LANES)]
                    tim = t_v[slot, b, pl.ds(EMBED_DIM + k * LANES, LANES)]
                    dre = hre * cosr - him * sinr - tre
                    dim = hre * sinr + him * cosr - tim
                    acc = acc + _sqrt16(dre * dre + dim * dim)
                accs.append(acc)
            for m in (1, 2, 4, 8):
                sel, perm = tree_sels[m], tree_perms[m]
                accs = [
                    lax.select(sel, accs[i + 1], accs[i])
                    + _shuffle(lax.select(sel, accs[i], accs[i + 1]), perm)
                    for i in range(0, len(accs), 2)
                ]
            out_v[pl.ds(j * CHUNK + g * LANES, LANES)] = accs[0]
            return _

        lax.fori_loop(0, CHUNK // LANES, group_body, None)

    pltpu.sync_copy(out_v, out_r.at[pl.ds(wid * B_PER_W, B_PER_W)])


@jax.jit
def _rotate_score(heads, relations, tails, entity_emb, relation_emb):
    cs_table = pl.pallas_call(
        _cs_table_body,
        out_shape=jax.ShapeDtypeStruct((NUM_RELS, ROW), jnp.float32),
    )(relation_emb)

    mesh = plsc.VectorSubcoreMesh(
        core_axis_name="c", subcore_axis_name="s",
        num_cores=NUM_CORES, num_subcores=NUM_SUBCORES)
    run = pl.kernel(
        _score_body,
        out_type=jax.ShapeDtypeStruct((BATCH,), jnp.float32),
        mesh=mesh,
        scratch_types=[
            pltpu.VMEM((N_CHUNKS, CHUNK), jnp.int32),
            pltpu.VMEM((N_CHUNKS, CHUNK), jnp.int32),
            pltpu.VMEM((N_CHUNKS, CHUNK), jnp.int32),
            pltpu.VMEM((2, CHUNK, ROW), jnp.float32),
            pltpu.VMEM((2, CHUNK, ROW), jnp.float32),
            pltpu.VMEM((2, CHUNK, ROW), jnp.float32),
            pltpu.VMEM((B_PER_W,), jnp.float32),
            pltpu.SemaphoreType.DMA,
            pltpu.SemaphoreType.DMA,
            pltpu.SemaphoreType.DMA,
        ],
    )
    return run(heads, relations, tails, entity_emb, cs_table)


def kernel(heads, relations, tails, entity_emb, relation_emb):
    shp = (NUM_WORKERS, N_CHUNKS, CHUNK)
    heads = heads.astype(jnp.int32).reshape(shp)
    relations = relations.astype(jnp.int32).reshape(shp)
    tails = tails.astype(jnp.int32).reshape(shp)
    return _rotate_score(heads, relations, tails, entity_emb, relation_emb)


# speedup vs baseline: 2.0687x; 1.0018x over previous
import jax
import jax.numpy as jnp
from jax import lax
from jax.experimental import pallas as pl
from jax.experimental.pallas import tpu as pltpu
from jax.experimental.pallas import tpu_sc as plsc

NUM_CORES = 2
NUM_SUBCORES = 16
NUM_WORKERS = NUM_CORES * NUM_SUBCORES
LANES = 16

BATCH = 16384
EMBED_DIM = 64
ROW = 2 * EMBED_DIM
NUM_RELS = 1000
B_PER_W = BATCH // NUM_WORKERS
CHUNK = 128
N_CHUNKS = B_PER_W // CHUNK


def _f32(x):
    return jnp.float32(x)


_GATHER_DNUMS = lax.GatherDimensionNumbers(
    offset_dims=(), collapsed_slice_dims=(0,), start_index_map=(0,))


def _shuffle(x, idx):
    return lax.gather(
        x, idx[:, None], dimension_numbers=_GATHER_DNUMS, slice_sizes=(1,),
        mode=lax.GatherScatterMode.PROMISE_IN_BOUNDS)


def _sqrt16(s):
    i = lax.bitcast_convert_type(s, jnp.int32)
    i = jnp.int32(0x5F3759DF) - lax.shift_right_logical(i, jnp.int32(1))
    y = lax.bitcast_convert_type(i, jnp.float32)
    y = y * (_f32(1.5) - _f32(0.5) * s * y * y)
    return s * y


def _cs_table_body(rel_ref, out_ref):
    r = rel_ref[...]
    out_ref[:, :EMBED_DIM] = jnp.cos(r)
    out_ref[:, EMBED_DIM:] = jnp.sin(r)


def _score_body(heads_r, rels_r, tails_r, ent_r, cs_r, out_r,
                hidx_v, ridx_v, tidx_v, h_v, t_v, r_v, out_v,
                sem_h, sem_t, sem_r):
    wid = lax.axis_index("s") * NUM_CORES + lax.axis_index("c")

    c1 = pltpu.async_copy(heads_r.at[wid], hidx_v, sem_h)
    c2 = pltpu.async_copy(rels_r.at[wid], ridx_v, sem_r)
    c3 = pltpu.async_copy(tails_r.at[wid], tidx_v, sem_t)
    c1.wait()
    c2.wait()
    c3.wait()

    def issue(j, slot):
        ch = pltpu.async_copy(ent_r.at[hidx_v.at[j]], h_v.at[slot], sem_h)
        ct = pltpu.async_copy(ent_r.at[tidx_v.at[j]], t_v.at[slot], sem_t)
        cr = pltpu.async_copy(cs_r.at[ridx_v.at[j]], r_v.at[slot], sem_r)
        return ch, ct, cr

    pending = issue(0, 0)
    lane = lax.iota(jnp.int32, LANES)
    tree_sels = {m: (lane & m) != 0 for m in (1, 2, 4, 8)}
    tree_perms = {m: lane ^ m for m in (1, 2, 4, 8)}

    for j in range(N_CHUNKS):
        slot = j & 1
        for c in pending:
            c.wait()
        if j + 1 < N_CHUNKS:
            pending = issue(j + 1, (j + 1) & 1)

        def group_body(g, _):
            accs = []
            for i in range(LANES):
                b = g * LANES + i
                acc = jnp.zeros((LANES,), jnp.float32)
                for k in range(EMBED_DIM // LANES):
                    cosr = r_v[slot, b, pl.ds(k * LANES, LANES)]
                    sinr = r_v[slot, b, pl.ds(EMBED_DIM + k * LANES, LANES)]
                    hre = h_v[slot, b, pl.ds(k * LANES, LANES)]
                    him = h_v[slot, b, pl.ds(EMBED_DIM + k * LANES, LANES)]
                    tre = t_v[slot, b, pl.ds(k * LANES, LANES)]
                    tim = t_v[slot, b, pl.ds(EMBED_DIM + k * LANES, LANES)]
                    dre = hre * cosr - him * sinr - tre
                    dim = hre * sinr + him * cosr - tim
                    acc = acc + _sqrt16(dre * dre + dim * dim)
                accs.append(acc)
            for m in (1, 2, 4, 8):
                sel, perm = tree_sels[m], tree_perms[m]
                accs = [
                    lax.select(sel, accs[i + 1], accs[i])
                    + _shuffle(lax.select(sel, accs[i], accs[i + 1]), perm)
                    for i in range(0, len(accs), 2)
                ]
            out_v[pl.ds(j * CHUNK + g * LANES, LANES)] = accs[0]
            return _

        lax.fori_loop(0, CHUNK // LANES, group_body, None)

    pltpu.sync_copy(out_v, out_r.at[pl.ds(wid * B_PER_W, B_PER_W)])


@jax.jit
def _rotate_score(heads, relations, tails, entity_emb, relation_emb):
    cs_table = pl.pallas_call(
        _cs_table_body,
        out_shape=jax.ShapeDtypeStruct((NUM_RELS, ROW), jnp.float32),
    )(relation_emb)

    mesh = plsc.VectorSubcoreMesh(
        core_axis_name="c", subcore_axis_name="s",
        num_cores=NUM_CORES, num_subcores=NUM_SUBCORES)
    run = pl.kernel(
        _score_body,
        out_type=jax.ShapeDtypeStruct((BATCH,), jnp.float32),
        mesh=mesh,
        scratch_types=[
            pltpu.VMEM((N_CHUNKS, CHUNK), jnp.int32),
            pltpu.VMEM((N_CHUNKS, CHUNK), jnp.int32),
            pltpu.VMEM((N_CHUNKS, CHUNK), jnp.int32),
            pltpu.VMEM((2, CHUNK, ROW), jnp.float32),
            pltpu.VMEM((2, CHUNK, ROW), jnp.float32),
            pltpu.VMEM((2, CHUNK, ROW), jnp.float32),
            pltpu.VMEM((B_PER_W,), jnp.float32),
            pltpu.SemaphoreType.DMA,
            pltpu.SemaphoreType.DMA,
            pltpu.SemaphoreType.DMA,
        ],
    )
    return run(heads, relations, tails, entity_emb, cs_table)


def kernel(heads, relations, tails, entity_emb, relation_emb):
    shp = (NUM_WORKERS, N_CHUNKS, CHUNK)
    heads = heads.astype(jnp.int32).reshape(shp)
    relations = relations.astype(jnp.int32).reshape(shp)
    tails = tails.astype(jnp.int32).reshape(shp)
    return _rotate_score(heads, relations, tails, entity_emb, relation_emb)
